# baseline (speedup 1.0000x reference)
import functools

import jax
import jax.numpy as jnp
from jax import lax
from jax.experimental import pallas as pl
from jax.experimental.pallas import tpu as pltpu
from jax.experimental.pallas import tpu_sc as plsc

_NC, _NS = 2, 16
_NW = _NC * _NS
_ROWS = 32


def kernel(inputs, pos_table):
    batch, seq_len, dim = inputs.shape
    seq_per_w = seq_len // _NW
    n_chunks = seq_per_w // _ROWS
    flat_in = inputs.reshape(batch * seq_len, dim)
    mesh = plsc.VectorSubcoreMesh(core_axis_name="c", subcore_axis_name="s")

    @functools.partial(
        pl.kernel,
        out_type=jax.ShapeDtypeStruct((batch * seq_len, dim), jnp.float32),
        mesh=mesh,
        scratch_types=[
            pltpu.VMEM((_ROWS, dim), jnp.float32),
            [pltpu.VMEM((_ROWS, dim), jnp.float32) for _ in range(4)],
            pltpu.SemaphoreType.DMA,
            [pltpu.SemaphoreType.DMA for _ in range(4)],
            [pltpu.SemaphoreType.DMA for _ in range(4)],
        ],
    )
    def sc_add(in_hbm, pos_hbm, out_hbm, pos_v, bufs, sem_pos, sems_in,
               sems_out):
        cid = lax.axis_index("c")
        sid = lax.axis_index("s")
        wid = sid * _NC + cid
        base = wid * seq_per_w

        def chunk(c, carry):
            row0 = base + c * _ROWS
            cp_pos = pltpu.async_copy(
                pos_hbm.at[pl.ds(row0, _ROWS)], pos_v, sem_pos)
            cp_in = [
                pltpu.async_copy(
                    in_hbm.at[pl.ds(b * seq_len + row0, _ROWS)], bufs[b],
                    sems_in[b])
                for b in range(batch)
            ]
            cp_out = [None] * batch
            cp_pos.wait()
            for b in range(batch):
                cp_in[b].wait()
                buf = bufs[b]

                @plsc.parallel_loop(0, _ROWS, unroll=4)
                def row_add(r):
                    for k in range(dim // 16):
                        x = pos_v[r, pl.ds(k * 16, 16)]
                        plsc.addupdate(buf.at[r, pl.ds(k * 16, 16)], x)

                cp_out[b] = pltpu.async_copy(
                    buf, out_hbm.at[pl.ds(b * seq_len + row0, _ROWS)],
                    sems_out[b])
            for b in range(batch):
                cp_out[b].wait()
            return carry

        lax.fori_loop(0, n_chunks, chunk, 0)

    out_flat = sc_add(flat_in, pos_table)
    return out_flat.reshape(batch, seq_len, dim)


# speedup vs baseline: 1.0861x; 1.0861x over previous
import functools

import jax
import jax.numpy as jnp
from jax import lax
from jax.experimental import pallas as pl
from jax.experimental.pallas import tpu as pltpu
from jax.experimental.pallas import tpu_sc as plsc

_NC, _NS = 2, 16
_NW = _NC * _NS
_ROWS = 32


def kernel(inputs, pos_table):
    batch, seq_len, dim = inputs.shape
    seq_per_w = seq_len // _NW
    n_chunks = seq_per_w // _ROWS
    flat_in = inputs.reshape(batch * seq_len, dim)
    mesh = plsc.VectorSubcoreMesh(core_axis_name="c", subcore_axis_name="s")

    @functools.partial(
        pl.kernel,
        out_type=jax.ShapeDtypeStruct((batch * seq_len, dim), jnp.float32),
        mesh=mesh,
        scratch_types=[
            pltpu.VMEM((_ROWS, dim), jnp.float32),
            pltpu.VMEM((_ROWS, dim), jnp.float32),
            pltpu.VMEM((_ROWS, dim), jnp.float32),
            pltpu.SemaphoreType.DMA,
            pltpu.SemaphoreType.DMA,
            pltpu.SemaphoreType.DMA,
            pltpu.SemaphoreType.DMA,
            pltpu.SemaphoreType.DMA,
        ],
    )
    def sc_add(in_hbm, pos_hbm, out_hbm, pos_v, buf0, buf1, sem_pos,
               sem_in0, sem_in1, sem_out0, sem_out1):
        cid = lax.axis_index("c")
        sid = lax.axis_index("s")
        wid = sid * _NC + cid
        base = wid * seq_per_w
        bufs = (buf0, buf1)
        sems_in = (sem_in0, sem_in1)
        sems_out = (sem_out0, sem_out1)

        def chunk(c, carry):
            row0 = base + c * _ROWS
            cp_pos = pltpu.async_copy(
                pos_hbm.at[pl.ds(row0, _ROWS)], pos_v, sem_pos)
            cp_in = [None] * batch
            cp_out = [None] * batch
            cp_in[0] = pltpu.async_copy(
                in_hbm.at[pl.ds(row0, _ROWS)], bufs[0], sems_in[0])
            cp_pos.wait()
            for b in range(batch):
                cur = b % 2
                nxt = 1 - cur
                cp_in[b].wait()
                if b + 1 < batch:
                    if b >= 1:
                        cp_out[b - 1].wait()
                    cp_in[b + 1] = pltpu.async_copy(
                        in_hbm.at[pl.ds((b + 1) * seq_len + row0, _ROWS)],
                        bufs[nxt], sems_in[nxt])
                buf = bufs[cur]

                @plsc.parallel_loop(0, _ROWS, unroll=4)
                def row_add(r):
                    for k in range(dim // 16):
                        x = pos_v[r, pl.ds(k * 16, 16)]
                        plsc.addupdate(buf.at[r, pl.ds(k * 16, 16)], x)

                cp_out[b] = pltpu.async_copy(
                    buf, out_hbm.at[pl.ds(b * seq_len + row0, _ROWS)],
                    sems_out[cur])
            cp_out[batch - 2].wait()
            cp_out[batch - 1].wait()
            return carry

        lax.fori_loop(0, n_chunks, chunk, 0)

    out_flat = sc_add(flat_in, pos_table)
    return out_flat.reshape(batch, seq_len, dim)


# speedup vs baseline: 1.1838x; 1.0900x over previous
import functools

import jax
import jax.numpy as jnp
from jax import lax
from jax.experimental import pallas as pl
from jax.experimental.pallas import tpu as pltpu
from jax.experimental.pallas import tpu_sc as plsc

_NC, _NS = 2, 16
_NW = _NC * _NS
_ROWS = 32
_NBUF = 3


def kernel(inputs, pos_table):
    batch, seq_len, dim = inputs.shape
    seq_per_w = seq_len // _NW
    n_chunks = seq_per_w // _ROWS
    flat_in = inputs.reshape(batch * seq_len, dim)
    mesh = plsc.VectorSubcoreMesh(core_axis_name="c", subcore_axis_name="s")

    @functools.partial(
        pl.kernel,
        out_type=jax.ShapeDtypeStruct((batch * seq_len, dim), jnp.float32),
        mesh=mesh,
        scratch_types=[
            pltpu.VMEM((_ROWS, dim), jnp.float32),
            [pltpu.VMEM((_ROWS, dim), jnp.float32) for _ in range(_NBUF)],
            pltpu.SemaphoreType.DMA,
            [pltpu.SemaphoreType.DMA for _ in range(_NBUF)],
            [pltpu.SemaphoreType.DMA for _ in range(_NBUF)],
        ],
    )
    def sc_add(in_hbm, pos_hbm, out_hbm, pos_v, bufs, sem_pos, sems_in,
               sems_out):
        cid = lax.axis_index("c")
        sid = lax.axis_index("s")
        wid = sid * _NC + cid
        base = wid * seq_per_w

        def chunk(c, carry):
            row0 = base + c * _ROWS
            cp_pos = pltpu.async_copy(
                pos_hbm.at[pl.ds(row0, _ROWS)], pos_v, sem_pos)
            cp_in = [None] * batch
            cp_out = [None] * batch
            cp_in[0] = pltpu.async_copy(
                in_hbm.at[pl.ds(row0, _ROWS)], bufs[0], sems_in[0])
            cp_in[1] = pltpu.async_copy(
                in_hbm.at[pl.ds(seq_len + row0, _ROWS)], bufs[1], sems_in[1])
            cp_pos.wait()
            for b in range(batch):
                cur = b % _NBUF
                cp_in[b].wait()
                if b + 2 < batch:
                    nxt = (b + 2) % _NBUF
                    if b >= 1:
                        cp_out[b - 1].wait()
                    cp_in[b + 2] = pltpu.async_copy(
                        in_hbm.at[pl.ds((b + 2) * seq_len + row0, _ROWS)],
                        bufs[nxt], sems_in[nxt])
                buf = bufs[cur]

                @plsc.parallel_loop(0, _ROWS)
                def row_add(r):
                    for k in range(dim // 16):
                        x = pos_v[r, pl.ds(k * 16, 16)]
                        plsc.addupdate(buf.at[r, pl.ds(k * 16, 16)], x)

                cp_out[b] = pltpu.async_copy(
                    buf, out_hbm.at[pl.ds(b * seq_len + row0, _ROWS)],
                    sems_out[cur])
            for b in range(1, batch):
                cp_out[b].wait()
            return carry

        lax.fori_loop(0, n_chunks, chunk, 0)

    out_flat = sc_add(flat_in, pos_table)
    return out_flat.reshape(batch, seq_len, dim)


# speedup vs baseline: 2.1166x; 1.7880x over previous
import jax
import jax.numpy as jnp
from jax.experimental import pallas as pl

_SEQ_BLOCK = 512


def _add_kernel(in_ref, pos_ref, out_ref):
    out_ref[...] = in_ref[...] + pos_ref[...][None, :, :]


def kernel(inputs, pos_table):
    batch, seq_len, dim = inputs.shape
    s_blk = _SEQ_BLOCK if seq_len % _SEQ_BLOCK == 0 else seq_len
    grid = (seq_len // s_blk,)
    return pl.pallas_call(
        _add_kernel,
        grid=grid,
        in_specs=[
            pl.BlockSpec((batch, s_blk, dim), lambda i: (0, i, 0)),
            pl.BlockSpec((s_blk, dim), lambda i: (i, 0)),
        ],
        out_specs=pl.BlockSpec((batch, s_blk, dim), lambda i: (0, i, 0)),
        out_shape=jax.ShapeDtypeStruct(inputs.shape, inputs.dtype),
    )(inputs, pos_table)
